# baseline (speedup 1.0000x reference)
import functools
import math

import jax
import jax.numpy as jnp
from jax import lax
from jax.experimental import pallas as pl
from jax.experimental.pallas import tpu as pltpu
from jax.experimental.pallas import tpu_sc as plsc

R = 16
N = 10000
E = 320000
D = 128
HID = 128
O = 16
NHEADS = 4
SEQ = 4
DH = D // NHEADS
NBINS = N * R

NC, NS = 2, 16
NW = NC * NS
EPW = E // NW
CH = 80
NCH = EPW // CH
DRT = 10
RPT = N // DRT
BPT = NBINS // NS

BN = 400
F32 = jnp.float32


def _mha_body(emb, wq, wk, wv, bq, bk, bv, wo, bo, root1, b1, w1cat,
              xr1_o, y1_o):
    col = lax.broadcasted_iota(jnp.int32, (D, NHEADS), 0) // DH
    hh = lax.broadcasted_iota(jnp.int32, (D, NHEADS), 1)
    G = (col == hh).astype(F32)

    q = jnp.dot(emb[0], wq[:], preferred_element_type=F32) + bq[:]
    qs = q * (1.0 / math.sqrt(DH))
    s_list = []
    v_list = []
    for m in range(SEQ):
        em = emb[m]
        km = jnp.dot(em, wk[:], preferred_element_type=F32) + bk[:]
        vm = jnp.dot(em, wv[:], preferred_element_type=F32) + bv[:]
        s_list.append(jnp.dot(qs * km, G, preferred_element_type=F32))
        v_list.append(vm)
    smax = s_list[0]
    for m in range(1, SEQ):
        smax = jnp.maximum(smax, s_list[m])
    p_list = [jnp.exp(s - smax) for s in s_list]
    z = p_list[0]
    for m in range(1, SEQ):
        z = z + p_list[m]
    o = None
    for m in range(SEQ):
        a = p_list[m] / z
        a_full = jnp.dot(a, G.T, preferred_element_type=F32)
        t = a_full * v_list[m]
        o = t if o is None else o + t
    x = jnp.dot(o, wo[:], preferred_element_type=F32) + bo[:]
    xr1_o[:] = jnp.dot(x, root1[:], preferred_element_type=F32) + b1[:]
    y1_o[:] = jnp.dot(x, w1cat[:], preferred_element_type=F32)


def _run_kernel_a(emb, wq, wk, wv, bq, bk, bv, wo, bo, root1, b1, w1cat):
    grid = (N // BN,)
    full = lambda shape: pl.BlockSpec(shape, lambda i: (0,) * len(shape))
    return pl.pallas_call(
        _mha_body,
        grid=grid,
        in_specs=[
            pl.BlockSpec((SEQ, BN, D), lambda i: (0, i, 0)),
            full((D, D)), full((D, D)), full((D, D)),
            full((1, D)), full((1, D)), full((1, D)),
            full((D, D)), full((1, D)),
            full((D, HID)), full((1, HID)),
            full((D, R * HID)),
        ],
        out_specs=[
            pl.BlockSpec((BN, HID), lambda i: (i, 0)),
            pl.BlockSpec((BN, R * HID), lambda i: (i, 0)),
        ],
        out_shape=[
            jax.ShapeDtypeStruct((N, HID), F32),
            jax.ShapeDtypeStruct((N, R * HID), F32),
        ],
    )(emb, wq, wk, wv, bq, bk, bv, wo, bo, root1, b1, w1cat)


def _layerb_body(xr1, p, w2cat, root2, b2, y2_o, xr2_o):
    h = jnp.maximum(xr1[:] + p[0] + p[1], 0.0)
    y2_o[:] = jnp.dot(h, w2cat[:], preferred_element_type=F32)
    xr2_o[:] = jnp.dot(h, root2[:], preferred_element_type=F32) + b2[:]


def _run_kernel_b(xr1, partials, w2cat, root2, b2):
    grid = (N // BN,)
    return pl.pallas_call(
        _layerb_body,
        grid=grid,
        in_specs=[
            pl.BlockSpec((BN, HID), lambda i: (i, 0)),
            pl.BlockSpec((NC, BN, HID), lambda i: (0, i, 0)),
            pl.BlockSpec((HID, R * O), lambda i: (0, 0)),
            pl.BlockSpec((HID, O), lambda i: (0, 0)),
            pl.BlockSpec((1, O), lambda i: (0, 0)),
        ],
        out_specs=[
            pl.BlockSpec((BN, R * O), lambda i: (i, 0)),
            pl.BlockSpec((BN, O), lambda i: (i, 0)),
        ],
        out_shape=[
            jax.ShapeDtypeStruct((N, R * O), F32),
            jax.ShapeDtypeStruct((N, O), F32),
        ],
    )(xr1, partials, w2cat, root2, b2)


def _final_body(xr2, p, out_o):
    t = xr2[:] + p[0] + p[1]
    out_o[:] = 1.0 / (1.0 + jnp.exp(-t))


def _run_kernel_c(xr2, partials):
    grid = (N // BN,)
    return pl.pallas_call(
        _final_body,
        grid=grid,
        in_specs=[
            pl.BlockSpec((BN, O), lambda i: (i, 0)),
            pl.BlockSpec((NC, BN, O), lambda i: (0, i, 0)),
        ],
        out_specs=pl.BlockSpec((BN, O), lambda i: (i, 0)),
        out_shape=jax.ShapeDtypeStruct((N, O), F32),
    )(xr2, partials)


@functools.cache
def _mesh():
    return plsc.VectorSubcoreMesh(core_axis_name="c", subcore_axis_name="s",
                                  num_cores=NC, num_subcores=NS)


HALF = NBINS // NC
HPT = HALF // NS


def _counts_body(eidx_hbm, inv_hbm, idx2_v, ones_v, zbuf_v, bins_sh, semw):
    c = lax.axis_index("c")
    s = lax.axis_index("s")

    pltpu.sync_copy(eidx_hbm.at[s], idx2_v)

    def zero16(i, _):
        zbuf_v[pl.ds(pl.multiple_of(i * 16, 8), 16)] = jnp.zeros((16,), F32)
        return 0

    lax.fori_loop(0, BPT // 16, zero16, 0)
    pltpu.sync_copy(zbuf_v, bins_sh.at[pl.ds(s * BPT, BPT)])
    for j in range(CH // 16):
        ones_v[pl.ds(j * 16, 16)] = jnp.ones((16,), F32)
    plsc.subcore_barrier()

    def fire(k, _):
        pltpu.async_copy(ones_v, bins_sh.at[idx2_v.at[k, 1]], semw, add=True)
        return 0

    def drain(k, _):
        pltpu.make_async_copy(ones_v, bins_sh.at[idx2_v.at[0, 1]],
                              semw).wait()
        return 0

    lax.fori_loop(0, NCH, fire, 0)
    lax.fori_loop(0, NCH, drain, 0)
    pltpu.sync_copy(eidx_hbm.at[NS + s], idx2_v)
    lax.fori_loop(0, NCH, fire, 0)
    lax.fori_loop(0, NCH, drain, 0)
    plsc.subcore_barrier()

    pltpu.sync_copy(bins_sh.at[pl.ds(c * HALF + s * HPT, HPT)],
                    zbuf_v.at[pl.ds(0, HPT)])

    def invstep(i, _):
        v = zbuf_v[pl.ds(i * 16, 16)]
        zbuf_v[pl.ds(i * 16, 16)] = 1.0 / jnp.maximum(v, 1.0)
        return 0

    lax.fori_loop(0, (HPT + 15) // 16, invstep, 0)
    pltpu.sync_copy(zbuf_v.at[pl.ds(0, HPT)],
                    inv_hbm.at[pl.ds(c * HALF + s * HPT, HPT)])


@functools.cache
def _counts_kernel():
    return functools.partial(
        pl.kernel,
        out_type=jax.ShapeDtypeStruct((NBINS,), F32),
        mesh=_mesh(),
        scratch_types=[
            pltpu.VMEM((NCH, 3, CH), jnp.int32),
            pltpu.VMEM((CH,), F32),
            pltpu.VMEM((BPT,), F32),
            pltpu.VMEM_SHARED((NBINS,), F32),
            pltpu.SemaphoreType.DMA,
        ],
    )(_counts_body)


SRT = 40
NBUF = 3
NFULL = NCH // NBUF
NTAIL = NCH - NFULL * NBUF


def _make_edge_body(width, scale_w):
    nsub = scale_w // 16

    def body(y_hbm, eidx_hbm, inv_hbm, out_hbm,
             idx3_v, scal_v, rows_v, stg_v, acc_sh, *sems):
        semi = sems[0:NBUF]
        semr = sems[NBUF:2 * NBUF]
        sems2 = sems[2 * NBUF:3 * NBUF]
        semw = sems[3 * NBUF:4 * NBUF]
        c = lax.axis_index("c")
        s = lax.axis_index("s")
        w = c * NS + s

        def zrow(r, _):
            for j in range(width // 16):
                stg_v[r, pl.ds(j * 16, 16)] = jnp.zeros((16,), F32)
            return 0

        lax.fori_loop(0, SRT, zrow, 0)

        @pl.when(s < DRT)
        def _():
            def zcopy(t, _):
                off = pl.multiple_of(s * RPT + t * SRT, 8)
                pltpu.sync_copy(stg_v, acc_sh.at[pl.ds(off, SRT)])
                return 0

            lax.fori_loop(0, RPT // SRT, zcopy, 0)

        plsc.subcore_barrier()

        def escale_chunk(b):
            def escale(g, _):
                sv = scal_v[b, pl.ds(pl.multiple_of(g * 16, 8), 16)]
                for j in range(16):
                    e = g * 16 + j
                    sc = sv[j]
                    for q in range(nsub):
                        rows_v[b, e, pl.ds(q * 16, 16)] = (
                            rows_v[b, e, pl.ds(q * 16, 16)] * sc)
                return 0

            lax.fori_loop(0, CH // 16, escale, 0)

        def process_block(base, nch):
            ids = [pltpu.async_copy(eidx_hbm.at[w, base + b], idx3_v.at[b],
                                    semi[b]) for b in range(nch)]
            gds = []
            for b in range(nch):
                ids[b].wait()
                g1 = pltpu.async_copy(y_hbm.at[idx3_v.at[b, 0]],
                                      rows_v.at[b], semr[b])
                g2 = pltpu.async_copy(inv_hbm.at[idx3_v.at[b, 1]],
                                      scal_v.at[b], sems2[b])
                gds.append((g1, g2))
            sds = []
            for b in range(nch):
                g1, g2 = gds[b]
                g2.wait()
                g1.wait()
                escale_chunk(b)
                sds.append(pltpu.async_copy(
                    rows_v.at[b], acc_sh.at[idx3_v.at[b, 2]],
                    semw[b], add=True))
            for d in sds:
                d.wait()

        def outer(ko, _):
            process_block(ko * NBUF, NBUF)
            return 0

        lax.fori_loop(0, NFULL, outer, 0)
        if NTAIL:
            process_block(NFULL * NBUF, NTAIL)
        plsc.subcore_barrier()

        @pl.when(s < DRT)
        def _():
            def drain(t, _):
                off = pl.multiple_of(s * RPT + t * SRT, 8)
                pltpu.sync_copy(acc_sh.at[pl.ds(off, SRT)], stg_v)
                pltpu.sync_copy(stg_v, out_hbm.at[pl.ds(c * N + off, SRT)])
                return 0

            lax.fori_loop(0, RPT // SRT, drain, 0)

    return body


@functools.cache
def _make_edge_kernel(width, scale_w):
    return functools.partial(
        pl.kernel,
        out_type=jax.ShapeDtypeStruct((NC * N, width), F32),
        mesh=_mesh(),
        scratch_types=[
            pltpu.VMEM((NBUF, 3, CH), jnp.int32),
            pltpu.VMEM((NBUF, CH), F32),
            pltpu.VMEM((NBUF, CH, width), F32),
            pltpu.VMEM((SRT, width), F32),
            pltpu.VMEM_SHARED((N, width), F32),
        ] + [pltpu.SemaphoreType.DMA] * (4 * NBUF),
    )(_make_edge_body(width, scale_w))


GR = CH * O // 128


def _edge2_body(y_hbm, idx_hbm, inv_hbm, out_hbm,
                idxb_v, scal_v, rows_v, zbuf_v, acc_sh, *sems):
    semi = sems[0:NBUF]
    semr = sems[NBUF:2 * NBUF]
    sems2 = sems[2 * NBUF:3 * NBUF]
    semw = sems[3 * NBUF:4 * NBUF]
    c = lax.axis_index("c")
    s = lax.axis_index("s")
    w = c * NS + s

    def zero16(i, _):
        zbuf_v[pl.ds(pl.multiple_of(i * 16, 8), 16)] = jnp.zeros((16,), F32)
        return 0

    lax.fori_loop(0, (N * O // NS) // 16, zero16, 0)
    pltpu.sync_copy(zbuf_v, acc_sh.at[pl.ds(s * (N * O // NS), N * O // NS)])
    plsc.subcore_barrier()

    def escale_chunk(b):
        def escale(g, _):
            sv = scal_v[b, pl.ds(pl.multiple_of(g * 16, 8), 16)]
            for j in range(16):
                r = 2 * g + j // 8
                col = (j % 8) * O
                sc = sv[j]
                rows_v[b, r, pl.ds(col, O)] = rows_v[b, r, pl.ds(col, O)] * sc
            return 0

        lax.fori_loop(0, CH // 16, escale, 0)

    def process_block(base, nch):
        ids = [pltpu.async_copy(idx_hbm.at[w, base + b], idxb_v.at[b],
                                semi[b]) for b in range(nch)]
        gds = []
        for b in range(nch):
            ids[b].wait()
            gs = [pltpu.async_copy(y_hbm.at[idxb_v.at[b, t]],
                                   rows_v.at[b, t], semr[b])
                  for t in range(GR)]
            gs.append(pltpu.async_copy(inv_hbm.at[idxb_v.at[b, 2 * GR]],
                                       scal_v.at[b], sems2[b]))
            gds.append(gs)
        sds = []
        for b in range(nch):
            for g in reversed(gds[b]):
                g.wait()
            escale_chunk(b)
            for t in range(GR):
                sds.append(pltpu.async_copy(
                    rows_v.at[b, t], acc_sh.at[idxb_v.at[b, GR + t]],
                    semw[b], add=True))
        for d in sds:
            d.wait()

    def outer(ko, _):
        process_block(ko * NBUF, NBUF)
        return 0

    lax.fori_loop(0, NFULL, outer, 0)
    if NTAIL:
        process_block(NFULL * NBUF, NTAIL)
    plsc.subcore_barrier()

    pltpu.sync_copy(acc_sh.at[pl.ds(s * (N * O // NS), N * O // NS)], zbuf_v)
    pltpu.sync_copy(zbuf_v, out_hbm.at[pl.ds(c * N * O + s * (N * O // NS),
                                             N * O // NS)])


@functools.cache
def _edge2_kernel():
    return functools.partial(
        pl.kernel,
        out_type=jax.ShapeDtypeStruct((NC * N * O,), F32),
        mesh=_mesh(),
        scratch_types=[
            pltpu.VMEM((NBUF, 2 * GR + 1, 128), jnp.int32),
            pltpu.VMEM((NBUF, 128), F32),
            pltpu.VMEM((NBUF, GR, 128), F32),
            pltpu.VMEM((N * O // NS,), F32),
            pltpu.VMEM_SHARED((N * O,), F32),
        ] + [pltpu.SemaphoreType.DMA] * (4 * NBUF),
    )(_edge2_body)


@jax.jit
def kernel(embedding, edge_index, edge_type, in_proj_w, in_proj_b,
           out_proj_w, out_proj_b, w1, root1, b1, w2, root2, b2):
    wq = in_proj_w[0:D].T
    wk = in_proj_w[D:2 * D].T
    wv = in_proj_w[2 * D:3 * D].T
    bq = in_proj_b[0:D].reshape(1, D)
    bk = in_proj_b[D:2 * D].reshape(1, D)
    bv = in_proj_b[2 * D:3 * D].reshape(1, D)
    wo = out_proj_w.T
    bo = out_proj_b.reshape(1, D)
    w1cat = w1.transpose(1, 0, 2).reshape(D, R * HID)
    w2cat = w2.transpose(1, 0, 2).reshape(HID, R * O)
    b1r = b1.reshape(1, HID)
    b2r = b2.reshape(1, O)

    src = edge_index[0]
    dst = edge_index[1]
    srcrel = src * R + edge_type
    dstrel = dst * R + edge_type
    eidx = jnp.stack([srcrel, dstrel, dst]).reshape(
        3, NW, NCH, CH).transpose(1, 2, 0, 3)
    lane = jnp.arange(O, dtype=jnp.int32)
    gidx = (srcrel[:, None] * O + lane).reshape(NW, NCH, GR, 128)
    sidx = (dst[:, None] * O + lane).reshape(NW, NCH, GR, 128)
    dpad = jnp.pad(dstrel.reshape(NW, NCH, 1, CH),
                   ((0, 0), (0, 0), (0, 0), (0, 128 - CH)))
    eidx2 = jnp.concatenate([gidx, sidx, dpad], axis=2)

    inv = _counts_kernel()(eidx)

    xr1, y1 = _run_kernel_a(embedding, wq, wk, wv, bq, bk, bv, wo, bo,
                            root1, b1r, w1cat)
    p1 = _make_edge_kernel(HID, HID)(y1.reshape(NBINS, HID), eidx,
                                     inv).reshape(NC, N, HID)
    y2, xr2 = _run_kernel_b(xr1, p1, w2cat, root2, b2r)
    p2 = _edge2_kernel()(y2.reshape(NBINS * O), eidx2,
                         inv).reshape(NC, N, O)
    return _run_kernel_c(xr2, p2)


# speedup vs baseline: 2.1925x; 2.1925x over previous
import functools
import math

import jax
import jax.numpy as jnp
from jax import lax
from jax.experimental import pallas as pl
from jax.experimental.pallas import tpu as pltpu
from jax.experimental.pallas import tpu_sc as plsc

R = 16
N = 10000
E = 320000
D = 128
HID = 128
O = 16
NHEADS = 4
SEQ = 4
DH = D // NHEADS
NBINS = N * R

NC, NS = 2, 16
NW = NC * NS
EPW = E // NW
CH = 80
NCH = EPW // CH
DRT = 10
RPT = N // DRT
BPT = NBINS // NS

BN = 400
F32 = jnp.float32


def _mha_body(emb, wq, wk, wv, bq, bk, bv, wo, bo, root1, b1, w1cat,
              xr1_o, y1_o):
    col = lax.broadcasted_iota(jnp.int32, (D, NHEADS), 0) // DH
    hh = lax.broadcasted_iota(jnp.int32, (D, NHEADS), 1)
    G = (col == hh).astype(F32)

    q = jnp.dot(emb[0], wq[:], preferred_element_type=F32) + bq[:]
    qs = q * (1.0 / math.sqrt(DH))
    s_list = []
    v_list = []
    for m in range(SEQ):
        em = emb[m]
        km = jnp.dot(em, wk[:], preferred_element_type=F32) + bk[:]
        vm = jnp.dot(em, wv[:], preferred_element_type=F32) + bv[:]
        s_list.append(jnp.dot(qs * km, G, preferred_element_type=F32))
        v_list.append(vm)
    smax = s_list[0]
    for m in range(1, SEQ):
        smax = jnp.maximum(smax, s_list[m])
    p_list = [jnp.exp(s - smax) for s in s_list]
    z = p_list[0]
    for m in range(1, SEQ):
        z = z + p_list[m]
    o = None
    for m in range(SEQ):
        a = p_list[m] / z
        a_full = jnp.dot(a, G.T, preferred_element_type=F32)
        t = a_full * v_list[m]
        o = t if o is None else o + t
    x = jnp.dot(o, wo[:], preferred_element_type=F32) + bo[:]
    xr1_o[:] = jnp.dot(x, root1[:], preferred_element_type=F32) + b1[:]
    y1_o[:] = jnp.dot(x, w1cat[:], preferred_element_type=F32)


def _run_kernel_a(emb, wq, wk, wv, bq, bk, bv, wo, bo, root1, b1, w1cat):
    grid = (N // BN,)
    full = lambda shape: pl.BlockSpec(shape, lambda i: (0,) * len(shape))
    return pl.pallas_call(
        _mha_body,
        grid=grid,
        in_specs=[
            pl.BlockSpec((SEQ, BN, D), lambda i: (0, i, 0)),
            full((D, D)), full((D, D)), full((D, D)),
            full((1, D)), full((1, D)), full((1, D)),
            full((D, D)), full((1, D)),
            full((D, HID)), full((1, HID)),
            full((D, R * HID)),
        ],
        out_specs=[
            pl.BlockSpec((BN, HID), lambda i: (i, 0)),
            pl.BlockSpec((BN, R * HID), lambda i: (i, 0)),
        ],
        out_shape=[
            jax.ShapeDtypeStruct((N, HID), F32),
            jax.ShapeDtypeStruct((N, R * HID), F32),
        ],
    )(emb, wq, wk, wv, bq, bk, bv, wo, bo, root1, b1, w1cat)


def _layerb_body(xr1, p, w2cat, root2, b2, y2_o, xr2_o):
    h = jnp.maximum(xr1[:] + p[0] + p[1], 0.0)
    y2_o[:] = jnp.dot(h, w2cat[:], preferred_element_type=F32)
    xr2_o[:] = jnp.dot(h, root2[:], preferred_element_type=F32) + b2[:]


def _run_kernel_b(xr1, partials, w2cat, root2, b2):
    grid = (N // BN,)
    return pl.pallas_call(
        _layerb_body,
        grid=grid,
        in_specs=[
            pl.BlockSpec((BN, HID), lambda i: (i, 0)),
            pl.BlockSpec((NC, BN, HID), lambda i: (0, i, 0)),
            pl.BlockSpec((HID, R * D), lambda i: (0, 0)),
            pl.BlockSpec((HID, O), lambda i: (0, 0)),
            pl.BlockSpec((1, O), lambda i: (0, 0)),
        ],
        out_specs=[
            pl.BlockSpec((BN, R * D), lambda i: (i, 0)),
            pl.BlockSpec((BN, O), lambda i: (i, 0)),
        ],
        out_shape=[
            jax.ShapeDtypeStruct((N, R * D), F32),
            jax.ShapeDtypeStruct((N, O), F32),
        ],
    )(xr1, partials, w2cat, root2, b2)


def _final_body(xr2, p, out_o):
    t = xr2[:] + p[0, :, 0:O] + p[1, :, 0:O]
    out_o[:] = 1.0 / (1.0 + jnp.exp(-t))


def _run_kernel_c(xr2, partials):
    grid = (N // BN,)
    return pl.pallas_call(
        _final_body,
        grid=grid,
        in_specs=[
            pl.BlockSpec((BN, O), lambda i: (i, 0)),
            pl.BlockSpec((NC, BN, D), lambda i: (0, i, 0)),
        ],
        out_specs=pl.BlockSpec((BN, O), lambda i: (i, 0)),
        out_shape=jax.ShapeDtypeStruct((N, O), F32),
    )(xr2, partials)


@functools.cache
def _mesh():
    return plsc.VectorSubcoreMesh(core_axis_name="c", subcore_axis_name="s",
                                  num_cores=NC, num_subcores=NS)


HALF = NBINS // NC
HPT = HALF // NS


def _counts_body(eidx_hbm, inv_hbm, idx2_v, ones_v, zbuf_v, bins_sh, semw):
    c = lax.axis_index("c")
    s = lax.axis_index("s")

    pltpu.sync_copy(eidx_hbm.at[s], idx2_v)

    def zero16(i, _):
        zbuf_v[pl.ds(pl.multiple_of(i * 16, 8), 16)] = jnp.zeros((16,), F32)
        return 0

    lax.fori_loop(0, BPT // 16, zero16, 0)
    pltpu.sync_copy(zbuf_v, bins_sh.at[pl.ds(s * BPT, BPT)])
    for j in range(CH // 16):
        ones_v[pl.ds(j * 16, 16)] = jnp.ones((16,), F32)
    plsc.subcore_barrier()

    def fire(k, _):
        pltpu.async_copy(ones_v, bins_sh.at[idx2_v.at[k, 1]], semw, add=True)
        return 0

    def drain(k, _):
        pltpu.make_async_copy(ones_v, bins_sh.at[idx2_v.at[0, 1]],
                              semw).wait()
        return 0

    lax.fori_loop(0, NCH, fire, 0)
    lax.fori_loop(0, NCH, drain, 0)
    pltpu.sync_copy(eidx_hbm.at[NS + s], idx2_v)
    lax.fori_loop(0, NCH, fire, 0)
    lax.fori_loop(0, NCH, drain, 0)
    plsc.subcore_barrier()

    pltpu.sync_copy(bins_sh.at[pl.ds(c * HALF + s * HPT, HPT)],
                    zbuf_v.at[pl.ds(0, HPT)])

    def invstep(i, _):
        v = zbuf_v[pl.ds(i * 16, 16)]
        zbuf_v[pl.ds(i * 16, 16)] = 1.0 / jnp.maximum(v, 1.0)
        return 0

    lax.fori_loop(0, (HPT + 15) // 16, invstep, 0)
    pltpu.sync_copy(zbuf_v.at[pl.ds(0, HPT)],
                    inv_hbm.at[pl.ds(c * HALF + s * HPT, HPT)])


@functools.cache
def _counts_kernel():
    return functools.partial(
        pl.kernel,
        out_type=jax.ShapeDtypeStruct((NBINS,), F32),
        mesh=_mesh(),
        scratch_types=[
            pltpu.VMEM((NCH, 3, CH), jnp.int32),
            pltpu.VMEM((CH,), F32),
            pltpu.VMEM((BPT,), F32),
            pltpu.VMEM_SHARED((NBINS,), F32),
            pltpu.SemaphoreType.DMA,
        ],
    )(_counts_body)


SRT = 40
NBUF = 3
NFULL = NCH // NBUF
NTAIL = NCH - NFULL * NBUF


def _make_edge_body(width, scale_w):
    nsub = scale_w // 16

    def body(y_hbm, eidx_hbm, inv_hbm, out_hbm,
             idx3_v, scal_v, rows_v, stg_v, acc_sh, *sems):
        semi = sems[0:NBUF]
        semr = sems[NBUF:2 * NBUF]
        sems2 = sems[2 * NBUF:3 * NBUF]
        semw = sems[3 * NBUF:4 * NBUF]
        c = lax.axis_index("c")
        s = lax.axis_index("s")
        w = c * NS + s

        def zrow(r, _):
            for j in range(width // 16):
                stg_v[r, pl.ds(j * 16, 16)] = jnp.zeros((16,), F32)
            return 0

        lax.fori_loop(0, SRT, zrow, 0)

        @pl.when(s < DRT)
        def _():
            def zcopy(t, _):
                off = pl.multiple_of(s * RPT + t * SRT, 8)
                pltpu.sync_copy(stg_v, acc_sh.at[pl.ds(off, SRT)])
                return 0

            lax.fori_loop(0, RPT // SRT, zcopy, 0)

        plsc.subcore_barrier()

        def escale_chunk(b):
            def escale(g, _):
                sv = scal_v[b, pl.ds(pl.multiple_of(g * 16, 8), 16)]
                for j in range(16):
                    e = g * 16 + j
                    sc = sv[j]
                    for q in range(nsub):
                        rows_v[b, e, pl.ds(q * 16, 16)] = (
                            rows_v[b, e, pl.ds(q * 16, 16)] * sc)
                return 0

            lax.fori_loop(0, CH // 16, escale, 0)

        def process_block(base, nch):
            ids = [pltpu.async_copy(eidx_hbm.at[w, base + b], idx3_v.at[b],
                                    semi[b]) for b in range(nch)]
            gds = []
            for b in range(nch):
                ids[b].wait()
                g1 = pltpu.async_copy(y_hbm.at[idx3_v.at[b, 0]],
                                      rows_v.at[b], semr[b])
                g2 = pltpu.async_copy(inv_hbm.at[idx3_v.at[b, 1]],
                                      scal_v.at[b], sems2[b])
                gds.append((g1, g2))
            sds = []
            for b in range(nch):
                g1, g2 = gds[b]
                g2.wait()
                g1.wait()
                escale_chunk(b)
                sds.append(pltpu.async_copy(
                    rows_v.at[b], acc_sh.at[idx3_v.at[b, 2]],
                    semw[b], add=True))
            for d in sds:
                d.wait()

        def outer(ko, _):
            process_block(ko * NBUF, NBUF)
            return 0

        lax.fori_loop(0, NFULL, outer, 0)
        if NTAIL:
            process_block(NFULL * NBUF, NTAIL)
        plsc.subcore_barrier()

        @pl.when(s < DRT)
        def _():
            def drain(t, _):
                off = pl.multiple_of(s * RPT + t * SRT, 8)
                pltpu.sync_copy(acc_sh.at[pl.ds(off, SRT)], stg_v)
                pltpu.sync_copy(stg_v, out_hbm.at[pl.ds(c * N + off, SRT)])
                return 0

            lax.fori_loop(0, RPT // SRT, drain, 0)

    return body


@functools.cache
def _make_edge_kernel(width, scale_w):
    return functools.partial(
        pl.kernel,
        out_type=jax.ShapeDtypeStruct((NC * N, width), F32),
        mesh=_mesh(),
        scratch_types=[
            pltpu.VMEM((NBUF, 3, CH), jnp.int32),
            pltpu.VMEM((NBUF, CH), F32),
            pltpu.VMEM((NBUF, CH, width), F32),
            pltpu.VMEM((SRT, width), F32),
            pltpu.VMEM_SHARED((N, width), F32),
        ] + [pltpu.SemaphoreType.DMA] * (4 * NBUF),
    )(_make_edge_body(width, scale_w))


@jax.jit
def kernel(embedding, edge_index, edge_type, in_proj_w, in_proj_b,
           out_proj_w, out_proj_b, w1, root1, b1, w2, root2, b2):
    wq = in_proj_w[0:D].T
    wk = in_proj_w[D:2 * D].T
    wv = in_proj_w[2 * D:3 * D].T
    bq = in_proj_b[0:D].reshape(1, D)
    bk = in_proj_b[D:2 * D].reshape(1, D)
    bv = in_proj_b[2 * D:3 * D].reshape(1, D)
    wo = out_proj_w.T
    bo = out_proj_b.reshape(1, D)
    w1cat = w1.transpose(1, 0, 2).reshape(D, R * HID)
    w2cat = jnp.pad(w2.transpose(1, 0, 2),
                    ((0, 0), (0, 0), (0, D - O))).reshape(HID, R * D)
    b1r = b1.reshape(1, HID)
    b2r = b2.reshape(1, O)

    src = edge_index[0]
    dst = edge_index[1]
    srcrel = src * R + edge_type
    dstrel = dst * R + edge_type
    eidx = jnp.stack([srcrel, dstrel, dst]).reshape(
        3, NW, NCH, CH).transpose(1, 2, 0, 3)

    inv = _counts_kernel()(eidx)

    xr1, y1 = _run_kernel_a(embedding, wq, wk, wv, bq, bk, bv, wo, bo,
                            root1, b1r, w1cat)
    p1 = _make_edge_kernel(HID, HID)(y1.reshape(NBINS, HID), eidx,
                                     inv).reshape(NC, N, HID)
    y2, xr2 = _run_kernel_b(xr1, p1, w2cat, root2, b2r)
    p2 = _make_edge_kernel(D, O)(y2.reshape(NBINS, D), eidx,
                                 inv).reshape(NC, N, D)
    return _run_kernel_c(xr2, p2)


# speedup vs baseline: 2.3410x; 1.0677x over previous
import functools
import math

import jax
import jax.numpy as jnp
from jax import lax
from jax.experimental import pallas as pl
from jax.experimental.pallas import tpu as pltpu
from jax.experimental.pallas import tpu_sc as plsc

R = 16
N = 10000
E = 320000
D = 128
HID = 128
O = 16
NHEADS = 4
SEQ = 4
DH = D // NHEADS
NBINS = N * R

NC, NS = 2, 16
NW = NC * NS
EPW = E // NW
CH = 80
NCH = EPW // CH
DRT = 10
RPT = N // DRT
BPT = NBINS // NS

BN = 1000
F32 = jnp.float32


def _mha_body(emb, wq, wk, wv, bq, bk, bv, wo, bo, root1, b1, w1cat,
              xr1_o, y1_o):
    col = lax.broadcasted_iota(jnp.int32, (D, NHEADS), 0) // DH
    hh = lax.broadcasted_iota(jnp.int32, (D, NHEADS), 1)
    G = (col == hh).astype(F32)

    q = jnp.dot(emb[0], wq[:], preferred_element_type=F32) + bq[:]
    qs = q * (1.0 / math.sqrt(DH))
    s_list = []
    v_list = []
    for m in range(SEQ):
        em = emb[m]
        km = jnp.dot(em, wk[:], preferred_element_type=F32) + bk[:]
        vm = jnp.dot(em, wv[:], preferred_element_type=F32) + bv[:]
        s_list.append(jnp.dot(qs * km, G, preferred_element_type=F32))
        v_list.append(vm)
    smax = s_list[0]
    for m in range(1, SEQ):
        smax = jnp.maximum(smax, s_list[m])
    p_list = [jnp.exp(s - smax) for s in s_list]
    z = p_list[0]
    for m in range(1, SEQ):
        z = z + p_list[m]
    o = None
    for m in range(SEQ):
        a = p_list[m] / z
        a_full = jnp.dot(a, G.T, preferred_element_type=F32)
        t = a_full * v_list[m]
        o = t if o is None else o + t
    x = jnp.dot(o, wo[:], preferred_element_type=F32) + bo[:]
    xr1_o[:] = jnp.dot(x, root1[:], preferred_element_type=F32) + b1[:]
    y1_o[:] = jnp.dot(x, w1cat[:], preferred_element_type=F32)


def _run_kernel_a(emb, wq, wk, wv, bq, bk, bv, wo, bo, root1, b1, w1cat):
    grid = (N // BN,)
    full = lambda shape: pl.BlockSpec(shape, lambda i: (0,) * len(shape))
    return pl.pallas_call(
        _mha_body,
        grid=grid,
        in_specs=[
            pl.BlockSpec((SEQ, BN, D), lambda i: (0, i, 0)),
            full((D, D)), full((D, D)), full((D, D)),
            full((1, D)), full((1, D)), full((1, D)),
            full((D, D)), full((1, D)),
            full((D, HID)), full((1, HID)),
            full((D, R * HID)),
        ],
        out_specs=[
            pl.BlockSpec((BN, HID), lambda i: (i, 0)),
            pl.BlockSpec((BN, R * HID), lambda i: (i, 0)),
        ],
        out_shape=[
            jax.ShapeDtypeStruct((N, HID), F32),
            jax.ShapeDtypeStruct((N, R * HID), F32),
        ],
    )(emb, wq, wk, wv, bq, bk, bv, wo, bo, root1, b1, w1cat)


def _layerb_body(xr1, p, w2cat, root2, b2, y2_o, xr2_o):
    h = jnp.maximum(xr1[:] + p[0] + p[1], 0.0)
    y2_o[:] = jnp.dot(h, w2cat[:], preferred_element_type=F32)
    xr2_o[:] = jnp.dot(h, root2[:], preferred_element_type=F32) + b2[:]


def _run_kernel_b(xr1, partials, w2cat, root2, b2):
    grid = (N // BN,)
    return pl.pallas_call(
        _layerb_body,
        grid=grid,
        in_specs=[
            pl.BlockSpec((BN, HID), lambda i: (i, 0)),
            pl.BlockSpec((NC, BN, HID), lambda i: (0, i, 0)),
            pl.BlockSpec((HID, R * D), lambda i: (0, 0)),
            pl.BlockSpec((HID, O), lambda i: (0, 0)),
            pl.BlockSpec((1, O), lambda i: (0, 0)),
        ],
        out_specs=[
            pl.BlockSpec((BN, R * D), lambda i: (i, 0)),
            pl.BlockSpec((BN, O), lambda i: (i, 0)),
        ],
        out_shape=[
            jax.ShapeDtypeStruct((N, R * D), F32),
            jax.ShapeDtypeStruct((N, O), F32),
        ],
    )(xr1, partials, w2cat, root2, b2)


def _final_body(xr2, p, out_o):
    t = xr2[:] + p[0, :, 0:O] + p[1, :, 0:O]
    out_o[:] = 1.0 / (1.0 + jnp.exp(-t))


def _run_kernel_c(xr2, partials):
    grid = (N // BN,)
    return pl.pallas_call(
        _final_body,
        grid=grid,
        in_specs=[
            pl.BlockSpec((BN, O), lambda i: (i, 0)),
            pl.BlockSpec((NC, BN, D), lambda i: (0, i, 0)),
        ],
        out_specs=pl.BlockSpec((BN, O), lambda i: (i, 0)),
        out_shape=jax.ShapeDtypeStruct((N, O), F32),
    )(xr2, partials)


@functools.cache
def _mesh():
    return plsc.VectorSubcoreMesh(core_axis_name="c", subcore_axis_name="s",
                                  num_cores=NC, num_subcores=NS)


HALF = NBINS // NC
HPT = HALF // NS


def _counts_body(eidx_hbm, inv_hbm, idx2_v, ones_v, zbuf_v, bins_sh, semw):
    c = lax.axis_index("c")
    s = lax.axis_index("s")

    pltpu.sync_copy(eidx_hbm.at[s], idx2_v)

    def zero16(i, _):
        zbuf_v[pl.ds(pl.multiple_of(i * 16, 8), 16)] = jnp.zeros((16,), F32)
        return 0

    lax.fori_loop(0, BPT // 16, zero16, 0)
    pltpu.sync_copy(zbuf_v, bins_sh.at[pl.ds(s * BPT, BPT)])
    for j in range(CH // 16):
        ones_v[pl.ds(j * 16, 16)] = jnp.ones((16,), F32)
    plsc.subcore_barrier()

    def fire(k, _):
        pltpu.async_copy(ones_v, bins_sh.at[idx2_v.at[k, 1]], semw, add=True)
        return 0

    def drain(k, _):
        pltpu.make_async_copy(ones_v, bins_sh.at[idx2_v.at[0, 1]],
                              semw).wait()
        return 0

    lax.fori_loop(0, NCH, fire, 0)
    lax.fori_loop(0, NCH, drain, 0)
    pltpu.sync_copy(eidx_hbm.at[NS + s], idx2_v)
    lax.fori_loop(0, NCH, fire, 0)
    lax.fori_loop(0, NCH, drain, 0)
    plsc.subcore_barrier()

    pltpu.sync_copy(bins_sh.at[pl.ds(c * HALF + s * HPT, HPT)],
                    zbuf_v.at[pl.ds(0, HPT)])

    def invstep(i, _):
        v = zbuf_v[pl.ds(i * 16, 16)]
        zbuf_v[pl.ds(i * 16, 16)] = 1.0 / jnp.maximum(v, 1.0)
        return 0

    lax.fori_loop(0, (HPT + 15) // 16, invstep, 0)
    pltpu.sync_copy(zbuf_v.at[pl.ds(0, HPT)],
                    inv_hbm.at[pl.ds(c * HALF + s * HPT, HPT)])


@functools.cache
def _counts_kernel():
    return functools.partial(
        pl.kernel,
        out_type=jax.ShapeDtypeStruct((NBINS,), F32),
        mesh=_mesh(),
        scratch_types=[
            pltpu.VMEM((NCH, 3, CH), jnp.int32),
            pltpu.VMEM((CH,), F32),
            pltpu.VMEM((BPT,), F32),
            pltpu.VMEM_SHARED((NBINS,), F32),
            pltpu.SemaphoreType.DMA,
        ],
    )(_counts_body)


SRT = 40
NBUF = 4
NFULL = NCH // NBUF
NTAIL = NCH - NFULL * NBUF


def _make_edge_body(width, scale_w):
    nsub = scale_w // 16

    def body(y_hbm, eidx_hbm, inv_hbm, out_hbm,
             idx3_v, scal_v, rows_v, stg_v, acc_sh, *sems):
        semi = sems[0:NBUF]
        semr = sems[NBUF:2 * NBUF]
        sems2 = sems[2 * NBUF:3 * NBUF]
        semw = sems[3 * NBUF:4 * NBUF]
        c = lax.axis_index("c")
        s = lax.axis_index("s")
        w = c * NS + s

        def zrow(r, _):
            for j in range(width // 16):
                stg_v[r, pl.ds(j * 16, 16)] = jnp.zeros((16,), F32)
            return 0

        lax.fori_loop(0, SRT, zrow, 0)

        @pl.when(s < DRT)
        def _():
            def zcopy(t, _):
                off = pl.multiple_of(s * RPT + t * SRT, 8)
                pltpu.sync_copy(stg_v, acc_sh.at[pl.ds(off, SRT)])
                return 0

            lax.fori_loop(0, RPT // SRT, zcopy, 0)

        plsc.subcore_barrier()

        def escale_chunk(b):
            def escale(g, _):
                sv = scal_v[b, pl.ds(pl.multiple_of(g * 16, 8), 16)]
                for j in range(16):
                    e = g * 16 + j
                    sc = sv[j]
                    for q in range(nsub):
                        rows_v[b, e, pl.ds(q * 16, 16)] = (
                            rows_v[b, e, pl.ds(q * 16, 16)] * sc)
                return 0

            lax.fori_loop(0, CH // 16, escale, 0)

        def process_block(base, nch):
            ids = [pltpu.async_copy(eidx_hbm.at[w, base + b], idx3_v.at[b],
                                    semi[b]) for b in range(nch)]
            gds = []
            for b in range(nch):
                ids[b].wait()
                g1 = pltpu.async_copy(y_hbm.at[idx3_v.at[b, 0]],
                                      rows_v.at[b], semr[b])
                g2 = pltpu.async_copy(inv_hbm.at[idx3_v.at[b, 1]],
                                      scal_v.at[b], sems2[b])
                gds.append((g1, g2))
            sds = []
            for b in range(nch):
                g1, g2 = gds[b]
                g2.wait()
                g1.wait()
                escale_chunk(b)
                sds.append(pltpu.async_copy(
                    rows_v.at[b], acc_sh.at[idx3_v.at[b, 2]],
                    semw[b], add=True))
            for d in sds:
                d.wait()

        def outer(ko, _):
            process_block(ko * NBUF, NBUF)
            return 0

        lax.fori_loop(0, NFULL, outer, 0)
        if NTAIL:
            process_block(NFULL * NBUF, NTAIL)
        plsc.subcore_barrier()

        @pl.when(s < DRT)
        def _():
            def drain(t, _):
                off = pl.multiple_of(s * RPT + t * SRT, 8)
                pltpu.sync_copy(acc_sh.at[pl.ds(off, SRT)], stg_v)
                pltpu.sync_copy(stg_v, out_hbm.at[pl.ds(c * N + off, SRT)])
                return 0

            lax.fori_loop(0, RPT // SRT, drain, 0)

    return body


@functools.cache
def _make_edge_kernel(width, scale_w):
    return functools.partial(
        pl.kernel,
        out_type=jax.ShapeDtypeStruct((NC * N, width), F32),
        mesh=_mesh(),
        scratch_types=[
            pltpu.VMEM((NBUF, 3, CH), jnp.int32),
            pltpu.VMEM((NBUF, CH), F32),
            pltpu.VMEM((NBUF, CH, width), F32),
            pltpu.VMEM((SRT, width), F32),
            pltpu.VMEM_SHARED((N, width), F32),
        ] + [pltpu.SemaphoreType.DMA] * (4 * NBUF),
    )(_make_edge_body(width, scale_w))


@jax.jit
def kernel(embedding, edge_index, edge_type, in_proj_w, in_proj_b,
           out_proj_w, out_proj_b, w1, root1, b1, w2, root2, b2):
    wq = in_proj_w[0:D].T
    wk = in_proj_w[D:2 * D].T
    wv = in_proj_w[2 * D:3 * D].T
    bq = in_proj_b[0:D].reshape(1, D)
    bk = in_proj_b[D:2 * D].reshape(1, D)
    bv = in_proj_b[2 * D:3 * D].reshape(1, D)
    wo = out_proj_w.T
    bo = out_proj_b.reshape(1, D)
    w1cat = w1.transpose(1, 0, 2).reshape(D, R * HID)
    w2cat = jnp.pad(w2.transpose(1, 0, 2),
                    ((0, 0), (0, 0), (0, D - O))).reshape(HID, R * D)
    b1r = b1.reshape(1, HID)
    b2r = b2.reshape(1, O)

    src = edge_index[0]
    dst = edge_index[1]
    srcrel = src * R + edge_type
    dstrel = dst * R + edge_type
    eidx = jnp.stack([srcrel, dstrel, dst]).reshape(
        3, NW, NCH, CH).transpose(1, 2, 0, 3)

    inv = _counts_kernel()(eidx)

    xr1, y1 = _run_kernel_a(embedding, wq, wk, wv, bq, bk, bv, wo, bo,
                            root1, b1r, w1cat)
    p1 = _make_edge_kernel(HID, HID)(y1.reshape(NBINS, HID), eidx,
                                     inv).reshape(NC, N, HID)
    y2, xr2 = _run_kernel_b(xr1, p1, w2cat, root2, b2r)
    p2 = _make_edge_kernel(D, O)(y2.reshape(NBINS, D), eidx,
                                 inv).reshape(NC, N, D)
    return _run_kernel_c(xr2, p2)
